# baseline (speedup 1.0000x reference)
import functools

import jax
import jax.numpy as jnp
from jax import lax
from jax.experimental import pallas as pl
from jax.experimental.pallas import tpu as pltpu
from jax.experimental.pallas import tpu_sc as plsc

N = 10000
E = 320000
D = 128
G = 64

NC = 2
NS = 16
NW = NC * NS

E_PER_C = E // NC
E_PER_W = E // NW
EK = 80
NCHK = E // EK
ECHUNKS = NCHK // NW
EREM = 0
NBUF = 3
PREF = 2
NIDX = 6
CH = 40
NCH = N // CH
PCH = 80
NPCH = N // PCH
PITER = -(-NPCH // NW)

_mesh = plsc.VectorSubcoreMesh(core_axis_name="c", subcore_axis_name="s")

_f32 = jnp.float32


def _zero_rows(buf, nrows):
    z16 = jnp.zeros((16,), _f32)

    def body(i, carry):
        for j in range(D // 16):
            buf[i, pl.ds(j * 16, 16)] = z16
        return carry

    lax.fori_loop(0, nrows, body, 0)


@functools.partial(
    pl.kernel,
    out_type=jax.ShapeDtypeStruct((NW, N // 16, 16), _f32),
    mesh=_mesh,
    compiler_params=pltpu.CompilerParams(needs_layout_passes=False),
    scratch_types=[
        pltpu.VMEM((E_PER_W,), jnp.int32),
        pltpu.VMEM((N // 16, 16), _f32),
    ],
)
def _sc_degree_hist(edge_hbm, hist_hbm, idx_v, hist_v):
    c = lax.axis_index("c")
    s = lax.axis_index("s")
    wid = c * NS + s

    z16 = jnp.zeros((16,), _f32)

    def zbody(i, carry):
        hist_v[i, :] = z16
        return carry

    lax.fori_loop(0, N // 16, zbody, 0)

    pltpu.sync_copy(edge_hbm.at[pl.ds(E + wid * E_PER_W, E_PER_W)], idx_v)
    ones = jnp.ones((16,), _f32)

    def body(i, carry):
        idx16 = idx_v[pl.ds(i * 16, 16)]
        plsc.addupdate_scatter(hist_v, [idx16 >> 4, idx16 & 15], ones)
        return carry

    lax.fori_loop(0, E_PER_W // 16, body, 0)
    pltpu.sync_copy(hist_v, hist_hbm.at[wid])


@functools.partial(
    pl.kernel,
    out_type=jax.ShapeDtypeStruct((NC, N, D), _f32),
    mesh=_mesh,
    compiler_params=pltpu.CompilerParams(needs_layout_passes=False),
    scratch_types=[
        [pltpu.VMEM((EK,), jnp.int32)] * NIDX,
        [pltpu.VMEM((EK,), jnp.int32)] * NIDX,
        [pltpu.VMEM((EK, D), _f32)] * NBUF,
        pltpu.VMEM_SHARED((N, D), _f32),
        [pltpu.SemaphoreType.DMA] * NIDX,
        [pltpu.SemaphoreType.DMA] * NBUF,
        [pltpu.SemaphoreType.DMA] * NBUF,
    ],
)
def _sc_edge_scatter(u_hbm, edge_hbm, out_hbm, sidx, didx, rows,
                     accum, isem, gsem, ssem):
    c = lax.axis_index("c")
    s = lax.axis_index("s")
    wid = c * NS + s

    def off(i):
        return wid * E_PER_W + i * EK

    def fire_idx(i, bi):
        pltpu.async_copy(edge_hbm.at[pl.ds(off(i), EK)], sidx[bi], isem[bi])
        pltpu.async_copy(edge_hbm.at[pl.ds(E + off(i), EK)], didx[bi],
                         isem[bi])

    def wait_idx(i, bi):
        pltpu.make_async_copy(edge_hbm.at[pl.ds(off(i), EK)], sidx[bi],
                              isem[bi]).wait()
        pltpu.make_async_copy(edge_hbm.at[pl.ds(E + off(i), EK)], didx[bi],
                              isem[bi]).wait()

    for j in range(NIDX - 1):
        fire_idx(j, j)

    zbuf = rows[0].at[pl.ds(0, CH)]
    _zero_rows(rows[0], EK)
    for k in range(-(-NCH // NS)):
        chunk = s + k * NS

        @pl.when(chunk < NCH)
        def _():
            pltpu.sync_copy(zbuf, accum.at[pl.ds(chunk * CH, CH)])

    plsc.subcore_barrier()

    for b in range(PREF):
        wait_idx(b, b)
        pltpu.async_copy(u_hbm.at[sidx[b]], rows[b], gsem[b])

    def visit(i, k):
        static = isinstance(i, int)
        b = k % NBUF
        bn = (b + PREF) % NBUF
        pltpu.make_async_copy(u_hbm.at[sidx[k]], rows[b], gsem[b]).wait()
        pltpu.async_copy(rows[b], accum.at[didx[k]], ssem[b], add=True)

        def wait_prev():
            pltpu.make_async_copy(rows[bn], accum.at[didx[(k - 1) % NIDX]],
                                  ssem[bn]).wait()

        def fire_next_idx():
            fire_idx(i + NIDX - 1, (k - 1) % NIDX)

        if static:
            if i >= 1:
                wait_prev()
            if i + NIDX - 1 < ECHUNKS:
                fire_next_idx()
        else:
            pl.when(i >= 1)(wait_prev)
            pl.when(i + NIDX - 1 < ECHUNKS)(fire_next_idx)

        wait_idx(i + PREF, (k + PREF) % NIDX)
        pltpu.async_copy(u_hbm.at[sidx[(k + PREF) % NIDX]], rows[bn], gsem[bn])

    def group(g, carry):
        for k in range(NIDX):
            visit(g * NIDX + k, k)
        return carry

    nloop = (ECHUNKS - PREF) // NIDX
    lax.fori_loop(0, nloop, group, 0)
    for i in range(nloop * NIDX, ECHUNKS - PREF):
        visit(i, i % NIDX)

    for j in range(PREF):
        i = ECHUNKS - PREF + j
        b = i % NBUF
        bn = (b + PREF) % NBUF
        pltpu.make_async_copy(u_hbm.at[sidx[i % NIDX]], rows[b],
                              gsem[b]).wait()
        pltpu.async_copy(rows[b], accum.at[didx[i % NIDX]], ssem[b], add=True)
        pltpu.make_async_copy(rows[bn], accum.at[didx[(i - 1) % NIDX]],
                              ssem[bn]).wait()
    i = ECHUNKS - 1
    pltpu.make_async_copy(rows[i % NBUF], accum.at[didx[i % NIDX]],
                          ssem[i % NBUF]).wait()

    plsc.subcore_barrier()
    for k in range(-(-NCH // NS)):
        chunk = s + k * NS

        @pl.when(chunk < NCH)
        def _():
            pltpu.sync_copy(accum.at[pl.ds(chunk * CH, CH)],
                            out_hbm.at[c, pl.ds(chunk * CH, CH)])


@functools.partial(
    pl.kernel,
    out_type=jax.ShapeDtypeStruct((NC, G, D), _f32),
    mesh=_mesh,
    compiler_params=pltpu.CompilerParams(needs_layout_passes=False),
    scratch_types=[
        pltpu.VMEM((PCH,), jnp.int32),
        pltpu.VMEM((PCH, D), _f32),
        pltpu.VMEM((8, D), _f32),
        pltpu.VMEM_SHARED((G, D), _f32),
    ],
)
def _sc_pool(y_hbm, batch_hbm, out_hbm, bidx, rows, zbuf, accum):
    c = lax.axis_index("c")
    s = lax.axis_index("s")
    wid = c * NS + s

    _zero_rows(zbuf, 8)

    @pl.when(s < G // 8)
    def _():
        pltpu.sync_copy(zbuf, accum.at[pl.ds(s * 8, 8)])

    plsc.subcore_barrier()

    def body(i, carry):
        chunk = wid + i * NW

        @pl.when(chunk < NPCH)
        def _():
            base = chunk * PCH
            pltpu.sync_copy(batch_hbm.at[pl.ds(base, PCH)], bidx)
            pltpu.sync_copy(y_hbm.at[pl.ds(base, PCH)], rows)
            pltpu.sync_copy(rows, accum.at[bidx], add=True)

        return carry

    lax.fori_loop(0, PITER, body, 0)
    plsc.subcore_barrier()

    @pl.when(s < G // 8)
    def _():
        pltpu.sync_copy(accum.at[pl.ds(s * 8, 8)], out_hbm.at[c, pl.ds(s * 8, 8)])


RB = 1000
NRB = N // RB


def _tc_stats_body(hist_ref, batch_ref, dinv_ref, cinv_ref):
    deg = jnp.sum(hist_ref[...], axis=0) + 1.0
    dinv_ref[...] = lax.rsqrt(deg)
    gids = lax.broadcasted_iota(jnp.int32, (1, G), 1)
    onehot = (batch_ref[...] == gids).astype(_f32)
    cnt = jnp.sum(onehot, axis=0, keepdims=True)
    cinv_ref[...] = 1.0 / jnp.maximum(cnt, 1.0)


_tc_stats = pl.pallas_call(
    _tc_stats_body,
    out_shape=[
        jax.ShapeDtypeStruct((N // 16, 16), _f32),
        jax.ShapeDtypeStruct((1, G), _f32),
    ],
)


def _tc_mm_scale_body(x_ref, w_ref, dinv_ref, o_ref):
    z = jnp.dot(x_ref[...], w_ref[...], preferred_element_type=_f32)
    o_ref[...] = dinv_ref[...] * z


_tc_mm_scale = pl.pallas_call(
    _tc_mm_scale_body,
    grid=(NRB,),
    in_specs=[
        pl.BlockSpec((RB, D), lambda i: (i, 0)),
        pl.BlockSpec((D, D), lambda i: (0, 0)),
        pl.BlockSpec((RB, 1), lambda i: (i, 0)),
    ],
    out_specs=pl.BlockSpec((RB, D), lambda i: (i, 0)),
    out_shape=jax.ShapeDtypeStruct((N, D), _f32),
)


def _tc_layer2_body(p_ref, u_ref, dinv_ref, b_ref, w_ref, o_ref):
    s = p_ref[0] + p_ref[1]
    h = jax.nn.relu(dinv_ref[...] * (s + u_ref[...]) + b_ref[...])
    o_ref[...] = dinv_ref[...] * jnp.dot(h, w_ref[...], preferred_element_type=_f32)


_tc_layer2 = pl.pallas_call(
    _tc_layer2_body,
    grid=(NRB,),
    in_specs=[
        pl.BlockSpec((NC, RB, D), lambda i: (0, i, 0)),
        pl.BlockSpec((RB, D), lambda i: (i, 0)),
        pl.BlockSpec((RB, 1), lambda i: (i, 0)),
        pl.BlockSpec((1, D), lambda i: (0, 0)),
        pl.BlockSpec((D, D), lambda i: (0, 0)),
    ],
    out_specs=pl.BlockSpec((RB, D), lambda i: (i, 0)),
    out_shape=jax.ShapeDtypeStruct((N, D), _f32),
)


def _tc_scale_sum_body(p_ref, u_ref, dinv_ref, o_ref):
    o_ref[...] = dinv_ref[...] * (p_ref[0] + p_ref[1] + u_ref[...])


_tc_scale_sum = pl.pallas_call(
    _tc_scale_sum_body,
    grid=(NRB,),
    in_specs=[
        pl.BlockSpec((NC, RB, D), lambda i: (0, i, 0)),
        pl.BlockSpec((RB, D), lambda i: (i, 0)),
        pl.BlockSpec((RB, 1), lambda i: (i, 0)),
    ],
    out_specs=pl.BlockSpec((RB, D), lambda i: (i, 0)),
    out_shape=jax.ShapeDtypeStruct((N, D), _f32),
)


def _tc_final_body(q_ref, cinv_ref, b_ref, o_ref):
    o_ref[...] = cinv_ref[...] * (q_ref[0] + q_ref[1]) + b_ref[...]


_tc_final = pl.pallas_call(
    _tc_final_body,
    in_specs=[
        pl.BlockSpec((NC, G, D), lambda: (0, 0, 0)),
        pl.BlockSpec((G, 1), lambda: (0, 0)),
        pl.BlockSpec((1, D), lambda: (0, 0)),
    ],
    out_specs=pl.BlockSpec((G, D), lambda: (0, 0)),
    out_shape=jax.ShapeDtypeStruct((G, D), _f32),
)


def kernel(x, edge_index, batch, W1, b1, W2, b2):
    edges = edge_index.astype(jnp.int32).reshape(2 * E)
    batch32 = batch.astype(jnp.int32)

    hist = _sc_degree_hist(edges)
    dinv2d, cinv1d = _tc_stats(hist, batch32.reshape(N, 1))
    dinv = dinv2d.reshape(N, 1)
    cinv = cinv1d.reshape(G, 1)

    u1 = _tc_mm_scale(x, W1, dinv)
    p1 = _sc_edge_scatter(u1, edges)
    u2 = _tc_layer2(p1, u1, dinv, b1.reshape(1, D), W2)
    p2 = _sc_edge_scatter(u2, edges)
    y = _tc_scale_sum(p2, u2, dinv)
    q = _sc_pool(y, batch32)
    out = _tc_final(q, cinv, b2.reshape(1, D))
    return out


# speedup vs baseline: 1.0140x; 1.0140x over previous
import functools

import jax
import jax.numpy as jnp
from jax import lax
from jax.experimental import pallas as pl
from jax.experimental.pallas import tpu as pltpu
from jax.experimental.pallas import tpu_sc as plsc

N = 10000
E = 320000
D = 128
G = 64

NC = 2
NS = 16
NW = NC * NS

E_PER_C = E // NC
E_PER_W = E // NW
EK = 80
NCHK = E // EK
ECHUNKS = NCHK // NW
EREM = 0
NBUF = 3
PREF = 2
NIDX = 6
CH = 40
NCH = N // CH
PCH = 80
NPCH = N // PCH
PITER = -(-NPCH // NW)

_mesh = plsc.VectorSubcoreMesh(core_axis_name="c", subcore_axis_name="s")

_f32 = jnp.float32


def _zero_rows(buf, nrows):
    z16 = jnp.zeros((16,), _f32)

    def body(i, carry):
        for j in range(D // 16):
            buf[i, pl.ds(j * 16, 16)] = z16
        return carry

    lax.fori_loop(0, nrows, body, 0)


@functools.partial(
    pl.kernel,
    out_type=jax.ShapeDtypeStruct((NW, N // 16, 16), _f32),
    mesh=_mesh,
    compiler_params=pltpu.CompilerParams(needs_layout_passes=False),
    scratch_types=[
        pltpu.VMEM((E_PER_W,), jnp.int32),
        pltpu.VMEM((N // 16, 16), _f32),
    ],
)
def _sc_degree_hist(edge_hbm, hist_hbm, idx_v, hist_v):
    c = lax.axis_index("c")
    s = lax.axis_index("s")
    wid = c * NS + s

    z16 = jnp.zeros((16,), _f32)

    def zbody(i, carry):
        hist_v[i, :] = z16
        return carry

    lax.fori_loop(0, N // 16, zbody, 0)

    pltpu.sync_copy(edge_hbm.at[pl.ds(E + wid * E_PER_W, E_PER_W)], idx_v)
    ones = jnp.ones((16,), _f32)

    def body(i, carry):
        idx16 = idx_v[pl.ds(i * 16, 16)]
        plsc.addupdate_scatter(hist_v, [idx16 >> 4, idx16 & 15], ones)
        return carry

    lax.fori_loop(0, E_PER_W // 16, body, 0)
    pltpu.sync_copy(hist_v, hist_hbm.at[wid])


@functools.partial(
    pl.kernel,
    out_type=jax.ShapeDtypeStruct((NC, N, D), _f32),
    mesh=_mesh,
    compiler_params=pltpu.CompilerParams(needs_layout_passes=False),
    scratch_types=[
        [pltpu.VMEM((EK,), jnp.int32)] * NIDX,
        [pltpu.VMEM((EK,), jnp.int32)] * NIDX,
        [pltpu.VMEM((EK, D), _f32)] * NBUF,
        pltpu.VMEM_SHARED((N, D), _f32),
        [pltpu.SemaphoreType.DMA] * NIDX,
        [pltpu.SemaphoreType.DMA] * NBUF,
        [pltpu.SemaphoreType.DMA] * NBUF,
    ],
)
def _sc_edge_scatter(u_hbm, edge_hbm, out_hbm, sidx, didx, rows,
                     accum, isem, gsem, ssem):
    c = lax.axis_index("c")
    s = lax.axis_index("s")
    wid = c * NS + s

    def off(i):
        return wid * E_PER_W + i * EK

    def fire_idx(i, bi):
        pltpu.async_copy(edge_hbm.at[pl.ds(off(i), EK)], sidx[bi], isem[bi])
        pltpu.async_copy(edge_hbm.at[pl.ds(E + off(i), EK)], didx[bi],
                         isem[bi])

    def wait_idx(i, bi):
        pltpu.make_async_copy(edge_hbm.at[pl.ds(off(i), EK)], sidx[bi],
                              isem[bi]).wait()
        pltpu.make_async_copy(edge_hbm.at[pl.ds(E + off(i), EK)], didx[bi],
                              isem[bi]).wait()

    for j in range(NIDX - 1):
        fire_idx(j, j)

    zbuf = rows[0].at[pl.ds(0, CH)]
    _zero_rows(rows[0], EK)
    for k in range(-(-NCH // NS)):
        chunk = s + k * NS

        @pl.when(chunk < NCH)
        def _():
            @pl.when(c == 0)
            def _():
                pltpu.sync_copy(u_hbm.at[pl.ds(chunk * CH, CH)],
                                accum.at[pl.ds(chunk * CH, CH)])

            @pl.when(c == 1)
            def _():
                pltpu.sync_copy(zbuf, accum.at[pl.ds(chunk * CH, CH)])

    plsc.subcore_barrier()

    for b in range(PREF):
        wait_idx(b, b)
        pltpu.async_copy(u_hbm.at[sidx[b]], rows[b], gsem[b])

    def visit(i, k):
        static = isinstance(i, int)
        b = k % NBUF
        bn = (b + PREF) % NBUF
        pltpu.make_async_copy(u_hbm.at[sidx[k]], rows[b], gsem[b]).wait()
        pltpu.async_copy(rows[b], accum.at[didx[k]], ssem[b], add=True)

        def wait_prev():
            pltpu.make_async_copy(rows[bn], accum.at[didx[(k - 1) % NIDX]],
                                  ssem[bn]).wait()

        def fire_next_idx():
            fire_idx(i + NIDX - 1, (k - 1) % NIDX)

        if static:
            if i >= 1:
                wait_prev()
            if i + NIDX - 1 < ECHUNKS:
                fire_next_idx()
        else:
            pl.when(i >= 1)(wait_prev)
            pl.when(i + NIDX - 1 < ECHUNKS)(fire_next_idx)

        wait_idx(i + PREF, (k + PREF) % NIDX)
        pltpu.async_copy(u_hbm.at[sidx[(k + PREF) % NIDX]], rows[bn], gsem[bn])

    def group(g, carry):
        for k in range(NIDX):
            visit(g * NIDX + k, k)
        return carry

    nloop = (ECHUNKS - PREF) // NIDX
    lax.fori_loop(0, nloop, group, 0)
    for i in range(nloop * NIDX, ECHUNKS - PREF):
        visit(i, i % NIDX)

    for j in range(PREF):
        i = ECHUNKS - PREF + j
        b = i % NBUF
        bn = (b + PREF) % NBUF
        pltpu.make_async_copy(u_hbm.at[sidx[i % NIDX]], rows[b],
                              gsem[b]).wait()
        pltpu.async_copy(rows[b], accum.at[didx[i % NIDX]], ssem[b], add=True)
        pltpu.make_async_copy(rows[bn], accum.at[didx[(i - 1) % NIDX]],
                              ssem[bn]).wait()
    i = ECHUNKS - 1
    pltpu.make_async_copy(rows[i % NBUF], accum.at[didx[i % NIDX]],
                          ssem[i % NBUF]).wait()

    plsc.subcore_barrier()
    for k in range(-(-NCH // NS)):
        chunk = s + k * NS

        @pl.when(chunk < NCH)
        def _():
            pltpu.sync_copy(accum.at[pl.ds(chunk * CH, CH)],
                            out_hbm.at[c, pl.ds(chunk * CH, CH)])


@functools.partial(
    pl.kernel,
    out_type=jax.ShapeDtypeStruct((NC, G, D), _f32),
    mesh=_mesh,
    compiler_params=pltpu.CompilerParams(needs_layout_passes=False),
    scratch_types=[
        pltpu.VMEM((PCH,), jnp.int32),
        pltpu.VMEM((PCH, D), _f32),
        pltpu.VMEM((8, D), _f32),
        pltpu.VMEM_SHARED((G, D), _f32),
    ],
)
def _sc_pool(y_hbm, batch_hbm, out_hbm, bidx, rows, zbuf, accum):
    c = lax.axis_index("c")
    s = lax.axis_index("s")
    wid = c * NS + s

    _zero_rows(zbuf, 8)

    @pl.when(s < G // 8)
    def _():
        pltpu.sync_copy(zbuf, accum.at[pl.ds(s * 8, 8)])

    plsc.subcore_barrier()

    def body(i, carry):
        chunk = wid + i * NW

        @pl.when(chunk < NPCH)
        def _():
            base = chunk * PCH
            pltpu.sync_copy(batch_hbm.at[pl.ds(base, PCH)], bidx)
            pltpu.sync_copy(y_hbm.at[pl.ds(base, PCH)], rows)
            pltpu.sync_copy(rows, accum.at[bidx], add=True)

        return carry

    lax.fori_loop(0, PITER, body, 0)
    plsc.subcore_barrier()

    @pl.when(s < G // 8)
    def _():
        pltpu.sync_copy(accum.at[pl.ds(s * 8, 8)], out_hbm.at[c, pl.ds(s * 8, 8)])


RB = 1000
NRB = N // RB


def _tc_stats_body(hist_ref, batch_ref, dinv_ref, cinv_ref):
    deg = jnp.sum(hist_ref[...], axis=0) + 1.0
    dinv_ref[...] = lax.rsqrt(deg)
    gids = lax.broadcasted_iota(jnp.int32, (1, G), 1)
    onehot = (batch_ref[...] == gids).astype(_f32)
    cnt = jnp.sum(onehot, axis=0, keepdims=True)
    cinv_ref[...] = 1.0 / jnp.maximum(cnt, 1.0)


_tc_stats = pl.pallas_call(
    _tc_stats_body,
    out_shape=[
        jax.ShapeDtypeStruct((N // 16, 16), _f32),
        jax.ShapeDtypeStruct((1, G), _f32),
    ],
)


def _tc_mm_scale_body(x_ref, w_ref, dinv_ref, o_ref):
    z = jnp.dot(x_ref[...], w_ref[...], preferred_element_type=_f32)
    o_ref[...] = dinv_ref[...] * z


_tc_mm_scale = pl.pallas_call(
    _tc_mm_scale_body,
    grid=(NRB,),
    in_specs=[
        pl.BlockSpec((RB, D), lambda i: (i, 0)),
        pl.BlockSpec((D, D), lambda i: (0, 0)),
        pl.BlockSpec((RB, 1), lambda i: (i, 0)),
    ],
    out_specs=pl.BlockSpec((RB, D), lambda i: (i, 0)),
    out_shape=jax.ShapeDtypeStruct((N, D), _f32),
)


def _tc_layer2_body(p_ref, dinv_ref, b_ref, w_ref, o_ref):
    h = jax.nn.relu(dinv_ref[...] * (p_ref[0] + p_ref[1]) + b_ref[...])
    o_ref[...] = dinv_ref[...] * jnp.dot(h, w_ref[...], preferred_element_type=_f32)


_tc_layer2 = pl.pallas_call(
    _tc_layer2_body,
    grid=(NRB,),
    in_specs=[
        pl.BlockSpec((NC, RB, D), lambda i: (0, i, 0)),
        pl.BlockSpec((RB, 1), lambda i: (i, 0)),
        pl.BlockSpec((1, D), lambda i: (0, 0)),
        pl.BlockSpec((D, D), lambda i: (0, 0)),
    ],
    out_specs=pl.BlockSpec((RB, D), lambda i: (i, 0)),
    out_shape=jax.ShapeDtypeStruct((N, D), _f32),
)


def _tc_scale_sum_body(p_ref, dinv_ref, o_ref):
    o_ref[...] = dinv_ref[...] * (p_ref[0] + p_ref[1])


_tc_scale_sum = pl.pallas_call(
    _tc_scale_sum_body,
    grid=(NRB,),
    in_specs=[
        pl.BlockSpec((NC, RB, D), lambda i: (0, i, 0)),
        pl.BlockSpec((RB, 1), lambda i: (i, 0)),
    ],
    out_specs=pl.BlockSpec((RB, D), lambda i: (i, 0)),
    out_shape=jax.ShapeDtypeStruct((N, D), _f32),
)


def _tc_final_body(q_ref, cinv_ref, b_ref, o_ref):
    o_ref[...] = cinv_ref[...] * (q_ref[0] + q_ref[1]) + b_ref[...]


_tc_final = pl.pallas_call(
    _tc_final_body,
    in_specs=[
        pl.BlockSpec((NC, G, D), lambda: (0, 0, 0)),
        pl.BlockSpec((G, 1), lambda: (0, 0)),
        pl.BlockSpec((1, D), lambda: (0, 0)),
    ],
    out_specs=pl.BlockSpec((G, D), lambda: (0, 0)),
    out_shape=jax.ShapeDtypeStruct((G, D), _f32),
)


def kernel(x, edge_index, batch, W1, b1, W2, b2):
    edges = edge_index.astype(jnp.int32).reshape(2 * E)
    batch32 = batch.astype(jnp.int32)

    hist = _sc_degree_hist(edges)
    dinv2d, cinv1d = _tc_stats(hist, batch32.reshape(N, 1))
    dinv = dinv2d.reshape(N, 1)
    cinv = cinv1d.reshape(G, 1)

    u1 = _tc_mm_scale(x, W1, dinv)
    p1 = _sc_edge_scatter(u1, edges)
    u2 = _tc_layer2(p1, dinv, b1.reshape(1, D), W2)
    p2 = _sc_edge_scatter(u2, edges)
    y = _tc_scale_sum(p2, dinv)
    q = _sc_pool(y, batch32)
    out = _tc_final(q, cinv, b2.reshape(1, D))
    return out


# speedup vs baseline: 1.0819x; 1.0670x over previous
import functools

import jax
import jax.numpy as jnp
from jax import lax
from jax.experimental import pallas as pl
from jax.experimental.pallas import tpu as pltpu
from jax.experimental.pallas import tpu_sc as plsc

N = 10000
E = 320000
D = 128
G = 64

NC = 2
NS = 16
NW = NC * NS

E_PER_C = E // NC
E_PER_W = E // NW
EK = 80
NCHK = E // EK
ECHUNKS = NCHK // NW
EREM = 0
NBUF = 3
PREF = 2
NIDX = 6
CH = 40
NCH = N // CH
PCH = 80
NPCH = N // PCH
PITER = -(-NPCH // NW)

_mesh = plsc.VectorSubcoreMesh(core_axis_name="c", subcore_axis_name="s")

_f32 = jnp.float32


def _zero_rows(buf, nrows):
    z16 = jnp.zeros((16,), _f32)

    def body(i, carry):
        for j in range(D // 16):
            buf[i, pl.ds(j * 16, 16)] = z16
        return carry

    lax.fori_loop(0, nrows, body, 0)


@functools.partial(
    pl.kernel,
    out_type=jax.ShapeDtypeStruct((NW, N // 16, 16), _f32),
    mesh=_mesh,
    compiler_params=pltpu.CompilerParams(needs_layout_passes=False),
    scratch_types=[
        pltpu.VMEM((E_PER_W,), jnp.int32),
        pltpu.VMEM((N // 16, 16), _f32),
    ],
)
def _sc_degree_hist(edge_hbm, hist_hbm, idx_v, hist_v):
    c = lax.axis_index("c")
    s = lax.axis_index("s")
    wid = c * NS + s

    z16 = jnp.zeros((16,), _f32)

    def zbody(i, carry):
        hist_v[i, :] = z16
        return carry

    lax.fori_loop(0, N // 16, zbody, 0)

    pltpu.sync_copy(edge_hbm.at[pl.ds(E + wid * E_PER_W, E_PER_W)], idx_v)
    ones = jnp.ones((16,), _f32)

    def body(i, carry):
        idx16 = idx_v[pl.ds(i * 16, 16)]
        plsc.addupdate_scatter(hist_v, [idx16 >> 4, idx16 & 15], ones)
        return carry

    lax.fori_loop(0, E_PER_W // 16, body, 0)
    pltpu.sync_copy(hist_v, hist_hbm.at[wid])


@functools.partial(
    pl.kernel,
    out_type=jax.ShapeDtypeStruct((NC, N, D), _f32),
    mesh=_mesh,
    compiler_params=pltpu.CompilerParams(needs_layout_passes=False),
    scratch_types=[
        [pltpu.VMEM((EK,), jnp.int32)] * NIDX,
        [pltpu.VMEM((EK,), jnp.int32)] * NIDX,
        [pltpu.VMEM((EK, D), _f32)] * NBUF,
        pltpu.VMEM_SHARED((N, D), _f32),
        [pltpu.SemaphoreType.DMA] * NIDX,
        [pltpu.SemaphoreType.DMA] * NBUF,
        [pltpu.SemaphoreType.DMA] * NBUF,
    ],
)
def _sc_edge_scatter(u_hbm, edge_hbm, out_hbm, sidx, didx, rows,
                     accum, isem, gsem, ssem):
    c = lax.axis_index("c")
    s = lax.axis_index("s")
    wid = c * NS + s

    def off(i):
        return wid * E_PER_W + i * EK

    def fire_idx(i, bi):
        pltpu.async_copy(edge_hbm.at[pl.ds(off(i), EK)], sidx[bi], isem[bi])
        pltpu.async_copy(edge_hbm.at[pl.ds(E + off(i), EK)], didx[bi],
                         isem[bi])

    def wait_idx(i, bi):
        pltpu.make_async_copy(edge_hbm.at[pl.ds(off(i), EK)], sidx[bi],
                              isem[bi]).wait()
        pltpu.make_async_copy(edge_hbm.at[pl.ds(E + off(i), EK)], didx[bi],
                              isem[bi]).wait()

    for j in range(NIDX - 1):
        fire_idx(j, j)

    zbuf = rows[0].at[pl.ds(0, CH)]
    _zero_rows(rows[0], EK)
    for k in range(-(-NCH // NS)):
        chunk = s + k * NS

        @pl.when(chunk < NCH)
        def _():
            pltpu.sync_copy(zbuf, accum.at[pl.ds(chunk * CH, CH)])

    plsc.subcore_barrier()

    for b in range(PREF):
        wait_idx(b, b)
        pltpu.async_copy(u_hbm.at[sidx[b]], rows[b], gsem[b])

    def visit(i, k):
        static = isinstance(i, int)
        b = k % NBUF
        bn = (b + PREF) % NBUF
        pltpu.make_async_copy(u_hbm.at[sidx[k]], rows[b], gsem[b]).wait()
        pltpu.async_copy(rows[b], accum.at[didx[k]], ssem[b], add=True)

        def wait_prev():
            pltpu.make_async_copy(rows[bn], accum.at[didx[(k - 1) % NIDX]],
                                  ssem[bn]).wait()

        def fire_next_idx():
            fire_idx(i + NIDX - 1, (k - 1) % NIDX)

        if static:
            if i >= 1:
                wait_prev()
            if i + NIDX - 1 < ECHUNKS:
                fire_next_idx()
        else:
            pl.when(i >= 1)(wait_prev)
            pl.when(i + NIDX - 1 < ECHUNKS)(fire_next_idx)

        wait_idx(i + PREF, (k + PREF) % NIDX)
        pltpu.async_copy(u_hbm.at[sidx[(k + PREF) % NIDX]], rows[bn], gsem[bn])

    def group(g, carry):
        for k in range(NIDX):
            visit(g * NIDX + k, k)
        return carry

    nloop = (ECHUNKS - PREF) // NIDX
    lax.fori_loop(0, nloop, group, 0)
    for i in range(nloop * NIDX, ECHUNKS - PREF):
        visit(i, i % NIDX)

    for j in range(PREF):
        i = ECHUNKS - PREF + j
        b = i % NBUF
        bn = (b + PREF) % NBUF
        pltpu.make_async_copy(u_hbm.at[sidx[i % NIDX]], rows[b],
                              gsem[b]).wait()
        pltpu.async_copy(rows[b], accum.at[didx[i % NIDX]], ssem[b], add=True)
        pltpu.make_async_copy(rows[bn], accum.at[didx[(i - 1) % NIDX]],
                              ssem[bn]).wait()
    i = ECHUNKS - 1
    pltpu.make_async_copy(rows[i % NBUF], accum.at[didx[i % NIDX]],
                          ssem[i % NBUF]).wait()

    plsc.subcore_barrier()
    for k in range(-(-NCH // NS)):
        chunk = s + k * NS

        @pl.when(chunk < NCH)
        def _():
            pltpu.sync_copy(accum.at[pl.ds(chunk * CH, CH)],
                            out_hbm.at[c, pl.ds(chunk * CH, CH)])


@functools.partial(
    pl.kernel,
    out_type=jax.ShapeDtypeStruct((NC, G, D), _f32),
    mesh=_mesh,
    compiler_params=pltpu.CompilerParams(needs_layout_passes=False),
    scratch_types=[
        pltpu.VMEM((PCH,), jnp.int32),
        pltpu.VMEM((PCH, D), _f32),
        pltpu.VMEM((8, D), _f32),
        pltpu.VMEM_SHARED((G, D), _f32),
    ],
)
def _sc_pool(y_hbm, batch_hbm, out_hbm, bidx, rows, zbuf, accum):
    c = lax.axis_index("c")
    s = lax.axis_index("s")
    wid = c * NS + s

    _zero_rows(zbuf, 8)

    @pl.when(s < G // 8)
    def _():
        pltpu.sync_copy(zbuf, accum.at[pl.ds(s * 8, 8)])

    plsc.subcore_barrier()

    def body(i, carry):
        chunk = wid + i * NW

        @pl.when(chunk < NPCH)
        def _():
            base = chunk * PCH
            pltpu.sync_copy(batch_hbm.at[pl.ds(base, PCH)], bidx)
            pltpu.sync_copy(y_hbm.at[pl.ds(base, PCH)], rows)
            pltpu.sync_copy(rows, accum.at[bidx], add=True)

        return carry

    lax.fori_loop(0, PITER, body, 0)
    plsc.subcore_barrier()

    @pl.when(s < G // 8)
    def _():
        pltpu.sync_copy(accum.at[pl.ds(s * 8, 8)], out_hbm.at[c, pl.ds(s * 8, 8)])


RB = 1000
NRB = N // RB


def _tc_stats_body(hist_ref, batch_ref, dinv_ref, cinv_ref):
    deg = jnp.sum(hist_ref[...], axis=0) + 1.0
    dinv_ref[...] = lax.rsqrt(deg)
    gids = lax.broadcasted_iota(jnp.int32, (1, G), 1)
    onehot = (batch_ref[...] == gids).astype(_f32)
    cnt = jnp.sum(onehot, axis=0, keepdims=True)
    cinv_ref[...] = 1.0 / jnp.maximum(cnt, 1.0)


_tc_stats = pl.pallas_call(
    _tc_stats_body,
    out_shape=[
        jax.ShapeDtypeStruct((N // 16, 16), _f32),
        jax.ShapeDtypeStruct((1, G), _f32),
    ],
)


def _tc_mm_scale_body(x_ref, w_ref, dinv_ref, o_ref):
    z = jnp.dot(x_ref[...], w_ref[...], preferred_element_type=_f32)
    o_ref[...] = dinv_ref[...] * z


_tc_mm_scale = pl.pallas_call(
    _tc_mm_scale_body,
    grid=(NRB,),
    in_specs=[
        pl.BlockSpec((RB, D), lambda i: (i, 0)),
        pl.BlockSpec((D, D), lambda i: (0, 0)),
        pl.BlockSpec((RB, 1), lambda i: (i, 0)),
    ],
    out_specs=pl.BlockSpec((RB, D), lambda i: (i, 0)),
    out_shape=jax.ShapeDtypeStruct((N, D), _f32),
)


def _tc_layer2_body(p_ref, u_ref, dinv_ref, b_ref, w_ref, o_ref):
    h = jax.nn.relu(dinv_ref[...] * (p_ref[0] + p_ref[1] + u_ref[...]) + b_ref[...])
    o_ref[...] = dinv_ref[...] * jnp.dot(h, w_ref[...], preferred_element_type=_f32)


_tc_layer2 = pl.pallas_call(
    _tc_layer2_body,
    grid=(NRB,),
    in_specs=[
        pl.BlockSpec((NC, RB, D), lambda i: (0, i, 0)),
        pl.BlockSpec((RB, D), lambda i: (i, 0)),
        pl.BlockSpec((RB, 1), lambda i: (i, 0)),
        pl.BlockSpec((1, D), lambda i: (0, 0)),
        pl.BlockSpec((D, D), lambda i: (0, 0)),
    ],
    out_specs=pl.BlockSpec((RB, D), lambda i: (i, 0)),
    out_shape=jax.ShapeDtypeStruct((N, D), _f32),
)


def _tc_scale_sum_body(p_ref, u_ref, dinv_ref, o_ref):
    o_ref[...] = dinv_ref[...] * (p_ref[0] + p_ref[1] + u_ref[...])


_tc_scale_sum = pl.pallas_call(
    _tc_scale_sum_body,
    grid=(NRB,),
    in_specs=[
        pl.BlockSpec((NC, RB, D), lambda i: (0, i, 0)),
        pl.BlockSpec((RB, D), lambda i: (i, 0)),
        pl.BlockSpec((RB, 1), lambda i: (i, 0)),
    ],
    out_specs=pl.BlockSpec((RB, D), lambda i: (i, 0)),
    out_shape=jax.ShapeDtypeStruct((N, D), _f32),
)


def _tc_final_body(q_ref, cinv_ref, b_ref, o_ref):
    o_ref[...] = cinv_ref[...] * (q_ref[0] + q_ref[1]) + b_ref[...]


_tc_final = pl.pallas_call(
    _tc_final_body,
    in_specs=[
        pl.BlockSpec((NC, G, D), lambda: (0, 0, 0)),
        pl.BlockSpec((G, 1), lambda: (0, 0)),
        pl.BlockSpec((1, D), lambda: (0, 0)),
    ],
    out_specs=pl.BlockSpec((G, D), lambda: (0, 0)),
    out_shape=jax.ShapeDtypeStruct((G, D), _f32),
)


def kernel(x, edge_index, batch, W1, b1, W2, b2):
    edges = edge_index.astype(jnp.int32).reshape(2 * E)
    batch32 = batch.astype(jnp.int32)

    hist = _sc_degree_hist(edges)
    dinv2d, cinv1d = _tc_stats(hist, batch32.reshape(N, 1))
    dinv = dinv2d.reshape(N, 1)
    cinv = cinv1d.reshape(G, 1)

    u1 = _tc_mm_scale(x, W1, dinv)
    p1 = _sc_edge_scatter(u1, edges)
    u2 = _tc_layer2(p1, u1, dinv, b1.reshape(1, D), W2)
    p2 = _sc_edge_scatter(u2, edges)
    y = _tc_scale_sum(p2, u2, dinv)
    q = _sc_pool(y, batch32)
    out = _tc_final(q, cinv, b2.reshape(1, D))
    return out
